# baseline (device time: 203704 ns/iter reference)
import jax
import jax.numpy as jnp
from jax import lax
from jax.experimental import pallas as pl
from jax.experimental.pallas import tpu as pltpu

N_DEV = 16
LOG_N = 4
ORDER_A = (0, 2, 1, 3)
ORDER_B = (2, 0, 3, 1)
STAGE_OFF = (0, 512, 768, 896)
SLOT_AG = 0
SLOT_XAG = 8
SLOT_RS = 12


def kernel(x, Win0, Wout0, Win1, Wout1, Win2, Wout2):
    B, D = x.shape
    H = Win0.shape[1]
    M = N_DEV * B
    D2 = D // 2

    def body(x_ref, win0_ref, wout0_ref, win1_ref, wout1_ref, win2_ref,
             wout2_ref, out_ref, acc_ref, out16_ref, stage_ref, send16_ref,
             wstage, ostage, win16, wout16, send_sems, recv_sems, copy_sems):
        me = lax.axis_index("i")

        barrier_sem = pltpu.get_barrier_semaphore()

        def xor_barrier():
            for k in range(LOG_N):
                pl.semaphore_signal(
                    barrier_sem, inc=1,
                    device_id=(me ^ (1 << k),),
                    device_id_type=pl.DeviceIdType.MESH,
                )
            pl.semaphore_wait(barrier_sem, LOG_N)

        def start_exchange(src, dst, partner, slot):
            rdma = pltpu.make_async_remote_copy(
                src_ref=src, dst_ref=dst,
                send_sem=send_sems.at[slot], recv_sem=recv_sems.at[slot],
                device_id=(partner,), device_id_type=pl.DeviceIdType.MESH,
            )
            rdma.start()
            return rdma

        def start_wload(l):
            wrefs = ((win0_ref, wout0_ref), (win1_ref, wout1_ref),
                     (win2_ref, wout2_ref))[l]
            pltpu.make_async_copy(wrefs[0], wstage, copy_sems.at[0]).start()
            pltpu.make_async_copy(wrefs[1], ostage, copy_sems.at[1]).start()

        def finish_wload(l):
            buf = l % 2
            pltpu.make_async_copy(win0_ref, wstage, copy_sems.at[0]).wait()
            pltpu.make_async_copy(wout0_ref, ostage, copy_sems.at[1]).wait()
            win16[buf] = wstage[:, :].astype(jnp.bfloat16)
            wout16[buf] = ostage[:, :].astype(jnp.bfloat16)

        cols = ((0, D2), (D2, D2))

        def all_reduce():
            los = [jnp.int32(0), jnp.int32(0)]
            for t in range(LOG_N):
                sz = M >> (t + 1)
                step = []
                for b, order in enumerate((ORDER_A, ORDER_B)):
                    kbit = order[t]
                    bit = (me >> kbit) & 1
                    keep_lo = los[b] + bit * sz
                    send_lo = los[b] + (1 - bit) * sz
                    c0, cw = cols[b]
                    srow = pl.ds(STAGE_OFF[t], sz)
                    cc = pl.ds(c0, cw)
                    send16_ref[srow, cc] = acc_ref[
                        pl.ds(send_lo, sz), cc].astype(jnp.bfloat16)
                    rdma = start_exchange(
                        send16_ref.at[srow, cc], stage_ref.at[srow, cc],
                        me ^ (1 << kbit), SLOT_RS + 4 * b + t,
                    )
                    step.append((rdma, keep_lo))
                for b, (rdma, keep_lo) in enumerate(step):
                    rdma.wait()
                    row = pl.ds(keep_lo, sz)
                    cc = pl.ds(cols[b][0], cols[b][1])
                    acc_ref[row, cc] = (
                        acc_ref[row, cc]
                        + stage_ref[pl.ds(STAGE_OFF[t], sz), cc].astype(
                            jnp.float32)
                    )
                    los[b] = keep_lo
            for b in range(2):
                cc = pl.ds(cols[b][0], cols[b][1])
                out16_ref[pl.ds(los[b], B), cc] = acc_ref[
                    pl.ds(los[b], B), cc].astype(jnp.bfloat16)
            for t in range(LOG_N - 1, -1, -1):
                sz = M >> (t + 1)
                step = []
                for b, order in enumerate((ORDER_A, ORDER_B)):
                    kbit = order[t]
                    bit = (me >> kbit) & 1
                    c0, cw = cols[b]
                    rdma = start_exchange(
                        out16_ref.at[pl.ds(los[b], sz), pl.ds(c0, cw)],
                        out16_ref.at[pl.ds(los[b], sz), pl.ds(c0, cw)],
                        me ^ (1 << kbit), SLOT_AG + 4 * b + t,
                    )
                    step.append((rdma, bit))
                for b, (rdma, bit) in enumerate(step):
                    rdma.wait()
                    los[b] = los[b] - bit * sz

        out16_ref[pl.ds(me * B, B), :] = x_ref[:, :].astype(jnp.bfloat16)
        start_wload(0)
        xor_barrier()
        lo = me * B
        for j in range(LOG_N):
            bs = B << j
            bit = (me >> j) & 1
            rdma = start_exchange(
                out16_ref.at[pl.ds(lo, bs), :],
                out16_ref.at[pl.ds(lo, bs), :], me ^ (1 << j), SLOT_XAG + j)
            rdma.wait()
            lo = lo - bit * bs

        finish_wload(0)
        start_wload(1)
        for l in range(3):
            buf = l % 2
            xg16 = out16_ref[:, :]
            hact = jnp.maximum(
                jnp.dot(xg16, win16[buf], preferred_element_type=jnp.float32),
                0.0,
            ).astype(jnp.bfloat16)
            acc_ref[:, :] = jnp.dot(
                hact, wout16[buf], preferred_element_type=jnp.float32
            )
            if l < 2:
                finish_wload(l + 1)
            if l == 0:
                start_wload(2)
            all_reduce()

        out_ref[:, :] = out16_ref[:, :].astype(jnp.float32)

    return pl.pallas_call(
        body,
        out_shape=jax.ShapeDtypeStruct((M, D), jnp.float32),
        in_specs=[pl.BlockSpec(memory_space=pltpu.VMEM)]
        + [pl.BlockSpec(memory_space=pl.ANY)] * 6,
        out_specs=pl.BlockSpec(memory_space=pltpu.VMEM),
        scratch_shapes=[
            pltpu.VMEM((M, D), jnp.float32),
            pltpu.VMEM((M, D), jnp.bfloat16),
            pltpu.VMEM((960, D), jnp.bfloat16),
            pltpu.VMEM((960, D), jnp.bfloat16),
            pltpu.VMEM((D, H), jnp.float32),
            pltpu.VMEM((H, D), jnp.float32),
            pltpu.VMEM((2, D, H), jnp.bfloat16),
            pltpu.VMEM((2, H, D), jnp.bfloat16),
            pltpu.SemaphoreType.DMA((20,)),
            pltpu.SemaphoreType.DMA((20,)),
            pltpu.SemaphoreType.DMA((2,)),
        ],
        compiler_params=pltpu.CompilerParams(
            collective_id=0,
            vmem_limit_bytes=100 * 1024 * 1024,
        ),
    )(x, Win0, Wout0, Win1, Wout1, Win2, Wout2)


# device time: 182772 ns/iter; 1.1145x vs baseline; 1.1145x over previous
import jax
import jax.numpy as jnp
from jax import lax
from jax.experimental import pallas as pl
from jax.experimental.pallas import tpu as pltpu

N_DEV = 16
LOG_N = 4
ORDER_A = (0, 2, 1, 3)
ORDER_B = (2, 0, 3, 1)
STAGE_OFF = (0, 512, 768, 896)
SLOT_AG = 0
SLOT_RS = 12
SLOT_XAG = 20


def kernel(x, Win0, Wout0, Win1, Wout1, Win2, Wout2):
    B, D = x.shape
    H = Win0.shape[1]
    M = N_DEV * B
    D2 = D // 2

    def body(x_ref, win0_ref, wout0_ref, win1_ref, wout1_ref, win2_ref,
             wout2_ref, out_ref, acc_ref, out16_ref, stage_ref, send16_ref,
             hact_ref, wstage, ostage, win16, wout16, send_sems, recv_sems,
             copy_sems):
        me = lax.axis_index("i")

        barrier_sem = pltpu.get_barrier_semaphore()

        def xor_barrier():
            for k in range(LOG_N):
                pl.semaphore_signal(
                    barrier_sem, inc=1,
                    device_id=(me ^ (1 << k),),
                    device_id_type=pl.DeviceIdType.MESH,
                )
            pl.semaphore_wait(barrier_sem, LOG_N)

        def start_exchange(src, dst, partner, slot):
            rdma = pltpu.make_async_remote_copy(
                src_ref=src, dst_ref=dst,
                send_sem=send_sems.at[slot], recv_sem=recv_sems.at[slot],
                device_id=(partner,), device_id_type=pl.DeviceIdType.MESH,
            )
            rdma.start()
            return rdma

        def start_wload(l):
            wrefs = ((win0_ref, wout0_ref), (win1_ref, wout1_ref),
                     (win2_ref, wout2_ref))[l]
            pltpu.make_async_copy(wrefs[0], wstage, copy_sems.at[0]).start()
            pltpu.make_async_copy(wrefs[1], ostage, copy_sems.at[1]).start()

        def finish_wload(l):
            buf = l % 2
            pltpu.make_async_copy(win0_ref, wstage, copy_sems.at[0]).wait()
            pltpu.make_async_copy(wout0_ref, ostage, copy_sems.at[1]).wait()
            win16[buf] = wstage[:, :].astype(jnp.bfloat16)
            wout16[buf] = ostage[:, :].astype(jnp.bfloat16)

        cols = ((0, D2), (D2, D2))

        def rs_step(t, los):
            sz = M >> (t + 1)
            step = []
            for b, order in enumerate((ORDER_A, ORDER_B)):
                kbit = order[t]
                bit = (me >> kbit) & 1
                keep_lo = los[b] + bit * sz
                send_lo = los[b] + (1 - bit) * sz
                c0, cw = cols[b]
                srow = pl.ds(STAGE_OFF[t], sz)
                cc = pl.ds(c0, cw)
                send16_ref[srow, cc] = acc_ref[
                    pl.ds(send_lo, sz), cc].astype(jnp.bfloat16)
                rdma = start_exchange(
                    send16_ref.at[srow, cc], stage_ref.at[srow, cc],
                    me ^ (1 << kbit), SLOT_RS + 4 * b + t,
                )
                step.append((rdma, keep_lo))
            new_los = []
            for b, (rdma, keep_lo) in enumerate(step):
                rdma.wait()
                row = pl.ds(keep_lo, sz)
                cc = pl.ds(cols[b][0], cols[b][1])
                acc_ref[row, cc] = (
                    acc_ref[row, cc]
                    + stage_ref[pl.ds(STAGE_OFF[t], sz), cc].astype(jnp.float32)
                )
                new_los.append(keep_lo)
            return new_los

        def ag_phase(los):
            for b in range(2):
                cc = pl.ds(cols[b][0], cols[b][1])
                out16_ref[pl.ds(los[b], B), cc] = acc_ref[
                    pl.ds(los[b], B), cc].astype(jnp.bfloat16)
            for t in range(LOG_N - 1, -1, -1):
                sz = M >> (t + 1)
                step = []
                for b, order in enumerate((ORDER_A, ORDER_B)):
                    kbit = order[t]
                    bit = (me >> kbit) & 1
                    c0, cw = cols[b]
                    rdma = start_exchange(
                        out16_ref.at[pl.ds(los[b], sz), pl.ds(c0, cw)],
                        out16_ref.at[pl.ds(los[b], sz), pl.ds(c0, cw)],
                        me ^ (1 << kbit), SLOT_AG + 4 * b + t,
                    )
                    step.append((rdma, bit))
                for b, (rdma, bit) in enumerate(step):
                    rdma.wait()
                    los[b] = los[b] - bit * sz

        out16_ref[pl.ds(me * B, B), :] = x_ref[:, :].astype(jnp.bfloat16)
        start_wload(0)
        xor_barrier()
        for j in range(LOG_N):
            pbit = 1 << (3 - j)
            partner = me ^ pbit
            base = me & ((1 << (4 - j)) - 1)
            rdmas = []
            for m in range(1 << j):
                c = base + (m << (4 - j))
                blk = out16_ref.at[pl.ds(c * B, B), :]
                rdmas.append(start_exchange(
                    blk, blk, partner, SLOT_XAG + (1 << j) - 1 + m))
            for rdma in rdmas:
                rdma.wait()

        finish_wload(0)
        start_wload(1)
        for l in range(3):
            buf = l % 2
            xg16 = out16_ref[:, :]
            for hc in range(2):
                hact_ref[:, pl.ds(hc * (H // 2), H // 2)] = jnp.maximum(
                    jnp.dot(xg16, win16[buf, :, pl.ds(hc * (H // 2), H // 2)],
                            preferred_element_type=jnp.float32),
                    0.0,
                ).astype(jnp.bfloat16)

            sz0 = M >> 1
            step1 = []
            for b, order in enumerate((ORDER_A, ORDER_B)):
                bit = (me >> order[0]) & 1
                keep_lo = bit * sz0
                send_lo = (1 - bit) * sz0
                c0, cw = cols[b]
                cc = pl.ds(c0, cw)
                acc_ref[pl.ds(send_lo, sz0), cc] = jnp.dot(
                    hact_ref[pl.ds(send_lo, sz0), :],
                    wout16[buf, :, cc], preferred_element_type=jnp.float32)
                srow = pl.ds(STAGE_OFF[0], sz0)
                send16_ref[srow, cc] = acc_ref[
                    pl.ds(send_lo, sz0), cc].astype(jnp.bfloat16)
                rdma = start_exchange(
                    send16_ref.at[srow, cc], stage_ref.at[srow, cc],
                    me ^ (1 << order[0]), SLOT_RS + 4 * b)
                step1.append((rdma, keep_lo))
            for b, (rdma, keep_lo) in enumerate(step1):
                c0, cw = cols[b]
                cc = pl.ds(c0, cw)
                acc_ref[pl.ds(keep_lo, sz0), cc] = jnp.dot(
                    hact_ref[pl.ds(keep_lo, sz0), :],
                    wout16[buf, :, cc], preferred_element_type=jnp.float32)
            if l < 2:
                finish_wload(l + 1)
            if l == 0:
                start_wload(2)

            los = []
            for b, (rdma, keep_lo) in enumerate(step1):
                rdma.wait()
                cc = pl.ds(cols[b][0], cols[b][1])
                row = pl.ds(keep_lo, sz0)
                acc_ref[row, cc] = (
                    acc_ref[row, cc]
                    + stage_ref[pl.ds(STAGE_OFF[0], sz0), cc].astype(
                        jnp.float32)
                )
                los.append(keep_lo)
            for t in range(1, LOG_N):
                los = rs_step(t, los)
            ag_phase(los)

        out_ref[:, :] = out16_ref[:, :].astype(jnp.float32)

    return pl.pallas_call(
        body,
        out_shape=jax.ShapeDtypeStruct((M, D), jnp.float32),
        in_specs=[pl.BlockSpec(memory_space=pltpu.VMEM)]
        + [pl.BlockSpec(memory_space=pl.ANY)] * 6,
        out_specs=pl.BlockSpec(memory_space=pltpu.VMEM),
        scratch_shapes=[
            pltpu.VMEM((M, D), jnp.float32),
            pltpu.VMEM((M, D), jnp.bfloat16),
            pltpu.VMEM((960, D), jnp.bfloat16),
            pltpu.VMEM((960, D), jnp.bfloat16),
            pltpu.VMEM((M, H), jnp.bfloat16),
            pltpu.VMEM((D, H), jnp.float32),
            pltpu.VMEM((H, D), jnp.float32),
            pltpu.VMEM((2, D, H), jnp.bfloat16),
            pltpu.VMEM((2, H, D), jnp.bfloat16),
            pltpu.SemaphoreType.DMA((35,)),
            pltpu.SemaphoreType.DMA((35,)),
            pltpu.SemaphoreType.DMA((2,)),
        ],
        compiler_params=pltpu.CompilerParams(
            collective_id=0,
            vmem_limit_bytes=100 * 1024 * 1024,
        ),
    )(x, Win0, Wout0, Win1, Wout1, Win2, Wout2)


# device time: 178750 ns/iter; 1.1396x vs baseline; 1.0225x over previous
import jax
import jax.numpy as jnp
from jax import lax
from jax.experimental import pallas as pl
from jax.experimental.pallas import tpu as pltpu

N_DEV = 16
LOG_N = 4
ORDER_A = (0, 2, 1, 3)
ORDER_B = (2, 0, 3, 1)
MB = 512
CH = MB // N_DEV
STAGE_OFF = ((0, 256, 384, 448), (512, 768, 896, 960))
SLOT_AG = 0
SLOT_RS = 12
SLOT_XAG = 20


def kernel(x, Win0, Wout0, Win1, Wout1, Win2, Wout2):
    B, D = x.shape
    H = Win0.shape[1]
    M = N_DEV * B

    def body(x_ref, win0_ref, wout0_ref, win1_ref, wout1_ref, win2_ref,
             wout2_ref, out_ref, acc_ref, out16_ref, stage_ref, send16_ref,
             hact_ref, wstage, ostage, win16, wout16, send_sems, recv_sems,
             copy_sems):
        me = lax.axis_index("i")

        barrier_sem = pltpu.get_barrier_semaphore()

        def xor_barrier():
            for k in range(LOG_N):
                pl.semaphore_signal(
                    barrier_sem, inc=1,
                    device_id=(me ^ (1 << k),),
                    device_id_type=pl.DeviceIdType.MESH,
                )
            pl.semaphore_wait(barrier_sem, LOG_N)

        def start_exchange(src, dst, partner, slot):
            rdma = pltpu.make_async_remote_copy(
                src_ref=src, dst_ref=dst,
                send_sem=send_sems.at[slot], recv_sem=recv_sems.at[slot],
                device_id=(partner,), device_id_type=pl.DeviceIdType.MESH,
            )
            rdma.start()
            return rdma

        def start_wload(l):
            wrefs = ((win0_ref, wout0_ref), (win1_ref, wout1_ref),
                     (win2_ref, wout2_ref))[l]
            pltpu.make_async_copy(wrefs[0], wstage, copy_sems.at[0]).start()
            pltpu.make_async_copy(wrefs[1], ostage, copy_sems.at[1]).start()

        def finish_wload(l):
            buf = l % 2
            pltpu.make_async_copy(win0_ref, wstage, copy_sems.at[0]).wait()
            pltpu.make_async_copy(wout0_ref, ostage, copy_sems.at[1]).wait()
            win16[buf] = wstage[:, :].astype(jnp.bfloat16)
            wout16[buf] = ostage[:, :].astype(jnp.bfloat16)

        def rs_step(t, los):
            sz = MB >> (t + 1)
            step = []
            for b, order in enumerate((ORDER_A, ORDER_B)):
                kbit = order[t]
                bit = (me >> kbit) & 1
                keep_lo = los[b] + bit * sz
                send_lo = los[b] + (1 - bit) * sz
                srow = pl.ds(STAGE_OFF[b][t], sz)
                send16_ref[srow, :] = acc_ref[
                    pl.ds(send_lo, sz), :].astype(jnp.bfloat16)
                rdma = start_exchange(
                    send16_ref.at[srow, :], stage_ref.at[srow, :],
                    me ^ (1 << kbit), SLOT_RS + 4 * b + t,
                )
                step.append((rdma, keep_lo))
            new_los = []
            for b, (rdma, keep_lo) in enumerate(step):
                rdma.wait()
                row = pl.ds(keep_lo, sz)
                acc_ref[row, :] = (
                    acc_ref[row, :]
                    + stage_ref[pl.ds(STAGE_OFF[b][t], sz), :].astype(
                        jnp.float32)
                )
                new_los.append(keep_lo)
            return new_los

        def ag_phase(los, consume):
            pending = []
            for b in range(2):
                out16_ref[pl.ds(los[b], CH), :] = acc_ref[
                    pl.ds(los[b], CH), :].astype(jnp.bfloat16)
                pending.append((los[b], CH))
            for t in range(LOG_N - 1, -1, -1):
                sz = MB >> (t + 1)
                step = []
                for b, order in enumerate((ORDER_A, ORDER_B)):
                    kbit = order[t]
                    bit = (me >> kbit) & 1
                    lo_p = los[b] + (1 - 2 * bit) * sz
                    rdma = start_exchange(
                        out16_ref.at[pl.ds(los[b], sz), :],
                        out16_ref.at[pl.ds(los[b], sz), :],
                        me ^ (1 << kbit), SLOT_AG + 4 * b + t,
                    )
                    step.append((rdma, bit, lo_p))
                for lo, s in pending:
                    consume(lo, s)
                pending = []
                for b, (rdma, bit, lo_p) in enumerate(step):
                    rdma.wait()
                    pending.append((lo_p, sz))
                    los[b] = los[b] - bit * sz
            for lo, s in pending:
                consume(lo, s)

        out16_ref[pl.ds(me * B, B), :] = x_ref[:, :].astype(jnp.bfloat16)
        start_wload(0)
        xor_barrier()
        for j in range(LOG_N):
            pbit = 1 << (3 - j)
            partner = me ^ pbit
            base = me & ((1 << (4 - j)) - 1)
            rdmas = []
            for m in range(1 << j):
                c = base + (m << (4 - j))
                blk = out16_ref.at[pl.ds(c * B, B), :]
                rdmas.append(start_exchange(
                    blk, blk, partner, SLOT_XAG + (1 << j) - 1 + m))
            for rdma in rdmas:
                rdma.wait()

        finish_wload(0)
        start_wload(1)
        xg16 = out16_ref[:, :]
        for hc in range(2):
            hact_ref[:, pl.ds(hc * (H // 2), H // 2)] = jnp.maximum(
                jnp.dot(xg16, win16[0, :, pl.ds(hc * (H // 2), H // 2)],
                        preferred_element_type=jnp.float32),
                0.0,
            ).astype(jnp.bfloat16)

        for l in range(3):
            buf = l % 2
            sz0 = MB >> 1
            step1 = []
            for b, order in enumerate((ORDER_A, ORDER_B)):
                bit = (me >> order[0]) & 1
                keep_lo = b * MB + bit * sz0
                send_lo = b * MB + (1 - bit) * sz0
                acc_ref[pl.ds(send_lo, sz0), :] = jnp.dot(
                    hact_ref[pl.ds(send_lo, sz0), :],
                    wout16[buf], preferred_element_type=jnp.float32)
                srow = pl.ds(STAGE_OFF[b][0], sz0)
                send16_ref[srow, :] = acc_ref[
                    pl.ds(send_lo, sz0), :].astype(jnp.bfloat16)
                rdma = start_exchange(
                    send16_ref.at[srow, :], stage_ref.at[srow, :],
                    me ^ (1 << order[0]), SLOT_RS + 4 * b)
                step1.append((rdma, keep_lo))
            for b, (rdma, keep_lo) in enumerate(step1):
                acc_ref[pl.ds(keep_lo, sz0), :] = jnp.dot(
                    hact_ref[pl.ds(keep_lo, sz0), :],
                    wout16[buf], preferred_element_type=jnp.float32)
            if l < 2:
                finish_wload(l + 1)
            if l == 0:
                start_wload(2)

            los = []
            for b, (rdma, keep_lo) in enumerate(step1):
                rdma.wait()
                row = pl.ds(keep_lo, sz0)
                acc_ref[row, :] = (
                    acc_ref[row, :]
                    + stage_ref[pl.ds(STAGE_OFF[b][0], sz0), :].astype(
                        jnp.float32)
                )
                los.append(keep_lo)
            for t in range(1, LOG_N):
                los = rs_step(t, los)

            if l < 2:
                nbuf = (l + 1) % 2

                def consume(lo, s, nbuf=nbuf):
                    hact_ref[pl.ds(lo, s), :] = jnp.maximum(
                        jnp.dot(out16_ref[pl.ds(lo, s), :], win16[nbuf],
                                preferred_element_type=jnp.float32),
                        0.0,
                    ).astype(jnp.bfloat16)
            else:

                def consume(lo, s):
                    out_ref[pl.ds(lo, s), :] = out16_ref[
                        pl.ds(lo, s), :].astype(jnp.float32)

            ag_phase(los, consume)

    return pl.pallas_call(
        body,
        out_shape=jax.ShapeDtypeStruct((M, D), jnp.float32),
        in_specs=[pl.BlockSpec(memory_space=pltpu.VMEM)]
        + [pl.BlockSpec(memory_space=pl.ANY)] * 6,
        out_specs=pl.BlockSpec(memory_space=pltpu.VMEM),
        scratch_shapes=[
            pltpu.VMEM((M, D), jnp.float32),
            pltpu.VMEM((M, D), jnp.bfloat16),
            pltpu.VMEM((1024, D), jnp.bfloat16),
            pltpu.VMEM((1024, D), jnp.bfloat16),
            pltpu.VMEM((M, H), jnp.bfloat16),
            pltpu.VMEM((D, H), jnp.float32),
            pltpu.VMEM((H, D), jnp.float32),
            pltpu.VMEM((2, D, H), jnp.bfloat16),
            pltpu.VMEM((2, H, D), jnp.bfloat16),
            pltpu.SemaphoreType.DMA((35,)),
            pltpu.SemaphoreType.DMA((35,)),
            pltpu.SemaphoreType.DMA((2,)),
        ],
        compiler_params=pltpu.CompilerParams(
            collective_id=0,
            vmem_limit_bytes=100 * 1024 * 1024,
        ),
    )(x, Win0, Wout0, Win1, Wout1, Win2, Wout2)


# device time: 176864 ns/iter; 1.1518x vs baseline; 1.0107x over previous
import jax
import jax.numpy as jnp
from jax import lax
from jax.experimental import pallas as pl
from jax.experimental.pallas import tpu as pltpu

N_DEV = 16
LOG_N = 4
ORDER_A = (0, 2, 1, 3)
ORDER_B = (2, 0, 3, 1)
MB = 512
CH = MB // N_DEV
STAGE_OFF = ((0, 256, 384, 448), (512, 768, 896, 960))
SLOT_AG = 0
SLOT_RS = 12
SLOT_XAG = 20


def kernel(x, Win0, Wout0, Win1, Wout1, Win2, Wout2):
    B, D = x.shape
    H = Win0.shape[1]
    M = N_DEV * B

    def body(x_ref, win0_ref, wout0_ref, win1_ref, wout1_ref, win2_ref,
             wout2_ref, out_ref, acc_ref, out16_ref, stage_ref, send16_ref,
             hact_ref, wstage, ostage, win16, wout16, send_sems, recv_sems,
             copy_sems):
        me = lax.axis_index("i")

        barrier_sem = pltpu.get_barrier_semaphore()

        def xor_barrier():
            for k in range(LOG_N):
                pl.semaphore_signal(
                    barrier_sem, inc=1,
                    device_id=(me ^ (1 << k),),
                    device_id_type=pl.DeviceIdType.MESH,
                )
            pl.semaphore_wait(barrier_sem, LOG_N)

        def start_exchange(src, dst, partner, slot):
            rdma = pltpu.make_async_remote_copy(
                src_ref=src, dst_ref=dst,
                send_sem=send_sems.at[slot], recv_sem=recv_sems.at[slot],
                device_id=(partner,), device_id_type=pl.DeviceIdType.MESH,
            )
            rdma.start()
            return rdma

        def start_wload(l):
            wrefs = ((win0_ref, wout0_ref), (win1_ref, wout1_ref),
                     (win2_ref, wout2_ref))[l]
            pltpu.make_async_copy(wrefs[0], wstage, copy_sems.at[0]).start()
            pltpu.make_async_copy(wrefs[1], ostage, copy_sems.at[1]).start()

        def finish_wload(l):
            buf = l % 2
            pltpu.make_async_copy(win0_ref, wstage, copy_sems.at[0]).wait()
            pltpu.make_async_copy(wout0_ref, ostage, copy_sems.at[1]).wait()
            win16[buf] = wstage[:, :].astype(jnp.bfloat16)
            wout16[buf] = ostage[:, :].astype(jnp.bfloat16)

        def rs_pipeline(step1):
            rdmas = [step1[0][0], step1[1][0]]
            keep = [step1[0][1], step1[1][1]]
            for t in range(LOG_N):
                sz = MB >> (t + 1)
                for b, order in enumerate((ORDER_A, ORDER_B)):
                    rdmas[b].wait()
                    row = pl.ds(keep[b], sz)
                    acc_ref[row, :] = (
                        acc_ref[row, :]
                        + stage_ref[pl.ds(STAGE_OFF[b][t], sz), :].astype(
                            jnp.float32)
                    )
                    if t < LOG_N - 1:
                        sz2 = MB >> (t + 2)
                        kbit = order[t + 1]
                        bit = (me >> kbit) & 1
                        keep2 = keep[b] + bit * sz2
                        send2 = keep[b] + (1 - bit) * sz2
                        srow = pl.ds(STAGE_OFF[b][t + 1], sz2)
                        send16_ref[srow, :] = acc_ref[
                            pl.ds(send2, sz2), :].astype(jnp.bfloat16)
                        rdmas[b] = start_exchange(
                            send16_ref.at[srow, :], stage_ref.at[srow, :],
                            me ^ (1 << kbit), SLOT_RS + 4 * b + t + 1,
                        )
                        keep[b] = keep2
            return keep

        def ag_phase(los, consume):
            pending = []
            rdmas = [None, None]
            meta = [None, None]
            for b, order in enumerate((ORDER_A, ORDER_B)):
                out16_ref[pl.ds(los[b], CH), :] = acc_ref[
                    pl.ds(los[b], CH), :].astype(jnp.bfloat16)
                pending.append((los[b], CH))
                kbit = order[LOG_N - 1]
                bit = (me >> kbit) & 1
                rdmas[b] = start_exchange(
                    out16_ref.at[pl.ds(los[b], CH), :],
                    out16_ref.at[pl.ds(los[b], CH), :],
                    me ^ (1 << kbit), SLOT_AG + 4 * b + LOG_N - 1,
                )
                meta[b] = (bit, los[b] + (1 - 2 * bit) * CH)
            for t in range(LOG_N - 1, -1, -1):
                sz = MB >> (t + 1)
                new_pending = []
                for b, order in enumerate((ORDER_A, ORDER_B)):
                    rdma, (bit, lo_p) = rdmas[b], meta[b]
                    rdma.wait()
                    new_pending.append((lo_p, sz))
                    los[b] = los[b] - bit * sz
                    if t > 0:
                        sz2 = MB >> t
                        kbit = order[t - 1]
                        bit2 = (me >> kbit) & 1
                        rdmas[b] = start_exchange(
                            out16_ref.at[pl.ds(los[b], sz2), :],
                            out16_ref.at[pl.ds(los[b], sz2), :],
                            me ^ (1 << kbit), SLOT_AG + 4 * b + t - 1,
                        )
                        meta[b] = (bit2, los[b] + (1 - 2 * bit2) * sz2)
                for lo, s in pending:
                    consume(lo, s)
                pending = new_pending
            for lo, s in pending:
                consume(lo, s)

        out16_ref[pl.ds(me * B, B), :] = x_ref[:, :].astype(jnp.bfloat16)
        start_wload(0)
        xor_barrier()
        for j in range(LOG_N):
            pbit = 1 << (3 - j)
            partner = me ^ pbit
            base = me & ((1 << (4 - j)) - 1)
            rdmas = []
            for m in range(1 << j):
                c = base + (m << (4 - j))
                blk = out16_ref.at[pl.ds(c * B, B), :]
                rdmas.append(start_exchange(
                    blk, blk, partner, SLOT_XAG + (1 << j) - 1 + m))
            for rdma in rdmas:
                rdma.wait()

        finish_wload(0)
        start_wload(1)
        xg16 = out16_ref[:, :]
        for hc in range(2):
            hact_ref[:, pl.ds(hc * (H // 2), H // 2)] = jnp.maximum(
                jnp.dot(xg16, win16[0, :, pl.ds(hc * (H // 2), H // 2)],
                        preferred_element_type=jnp.float32),
                0.0,
            ).astype(jnp.bfloat16)

        for l in range(3):
            buf = l % 2
            sz0 = MB >> 1
            step1 = []
            for b, order in enumerate((ORDER_A, ORDER_B)):
                bit = (me >> order[0]) & 1
                keep_lo = b * MB + bit * sz0
                send_lo = b * MB + (1 - bit) * sz0
                acc_ref[pl.ds(send_lo, sz0), :] = jnp.dot(
                    hact_ref[pl.ds(send_lo, sz0), :],
                    wout16[buf], preferred_element_type=jnp.float32)
                srow = pl.ds(STAGE_OFF[b][0], sz0)
                send16_ref[srow, :] = acc_ref[
                    pl.ds(send_lo, sz0), :].astype(jnp.bfloat16)
                rdma = start_exchange(
                    send16_ref.at[srow, :], stage_ref.at[srow, :],
                    me ^ (1 << order[0]), SLOT_RS + 4 * b)
                step1.append((rdma, keep_lo))
            for b, (rdma, keep_lo) in enumerate(step1):
                acc_ref[pl.ds(keep_lo, sz0), :] = jnp.dot(
                    hact_ref[pl.ds(keep_lo, sz0), :],
                    wout16[buf], preferred_element_type=jnp.float32)
            if l < 2:
                finish_wload(l + 1)
            if l == 0:
                start_wload(2)

            los = rs_pipeline(step1)

            if l < 2:
                nbuf = (l + 1) % 2

                def consume(lo, s, nbuf=nbuf):
                    hact_ref[pl.ds(lo, s), :] = jnp.maximum(
                        jnp.dot(out16_ref[pl.ds(lo, s), :], win16[nbuf],
                                preferred_element_type=jnp.float32),
                        0.0,
                    ).astype(jnp.bfloat16)
            else:

                def consume(lo, s):
                    out_ref[pl.ds(lo, s), :] = out16_ref[
                        pl.ds(lo, s), :].astype(jnp.float32)

            ag_phase(los, consume)

    return pl.pallas_call(
        body,
        out_shape=jax.ShapeDtypeStruct((M, D), jnp.float32),
        in_specs=[pl.BlockSpec(memory_space=pltpu.VMEM)]
        + [pl.BlockSpec(memory_space=pl.ANY)] * 6,
        out_specs=pl.BlockSpec(memory_space=pltpu.VMEM),
        scratch_shapes=[
            pltpu.VMEM((M, D), jnp.float32),
            pltpu.VMEM((M, D), jnp.bfloat16),
            pltpu.VMEM((1024, D), jnp.bfloat16),
            pltpu.VMEM((1024, D), jnp.bfloat16),
            pltpu.VMEM((M, H), jnp.bfloat16),
            pltpu.VMEM((D, H), jnp.float32),
            pltpu.VMEM((H, D), jnp.float32),
            pltpu.VMEM((2, D, H), jnp.bfloat16),
            pltpu.VMEM((2, H, D), jnp.bfloat16),
            pltpu.SemaphoreType.DMA((35,)),
            pltpu.SemaphoreType.DMA((35,)),
            pltpu.SemaphoreType.DMA((2,)),
        ],
        compiler_params=pltpu.CompilerParams(
            collective_id=0,
            vmem_limit_bytes=100 * 1024 * 1024,
        ),
    )(x, Win0, Wout0, Win1, Wout1, Win2, Wout2)
